# Initial kernel scaffold; baseline (speedup 1.0000x reference)
import jax
import jax.numpy as jnp
from jax.experimental import pallas as pl


def kernel(x, edge_index, W1_l, W1_r, b1, Wmu_l, Wmu_r, bmu, Wlv_l, Wlv_r, blv, Wdec, bdec):
    raise NotImplementedError("write your pallas kernel here")


# speedup vs baseline: 3.5759x; 3.5759x over previous
import functools

import jax
import jax.numpy as jnp
from jax import lax
from jax.experimental import pallas as pl
from jax.experimental.pallas import tpu as pltpu
from jax.experimental.pallas import tpu_sc as plsc

N = 10000
E = 320000
D_IN = 128
D_H = 128
D_Z = 64

NC = 2
NS = 16
CH = 80
EPW = E // (NC * NS)
STEPS = EPW // CH
EPH = E // NS
HSTEPS = EPH // CH
S_LO = 624
S_HI = N - S_LO * (NS - 1)


def _sc_agg_body(table, src, dst, z128, out,
                 src_v, dst_v, dst2_v, rows_v, ones_v, zbuf, dbuf, acc, dacc, sem):
    c = lax.axis_index("c")
    s = lax.axis_index("s")

    def for_stripe(fn):
        @pl.when(s < NS - 1)
        def _lo():
            fn(pl.multiple_of(s * S_LO, 8), S_LO)

        @pl.when(s == NS - 1)
        def _hi():
            fn((NS - 1) * S_LO, S_HI)

    zeros16 = jnp.zeros((16,), jnp.float32)
    ones16 = jnp.ones((16,), jnp.float32)
    for j in range(16):
        zbuf[j, :] = zeros16
    for j in range(CH):
        ones_v[j, :] = ones16

    def zero(o, sz):
        pltpu.sync_copy(z128.at[pl.ds(o, sz)], acc.at[pl.ds(o, sz)])
        for k in range(sz // 16):
            pltpu.sync_copy(zbuf, dacc.at[pl.ds(pl.multiple_of(o + k * 16, 8), 16)])

    for_stripe(zero)
    plsc.subcore_barrier()

    ebase = (c * NS + s) * EPW

    def step(i, carry):
        off = pl.multiple_of(ebase + i * CH, 8)
        pltpu.sync_copy(src.at[pl.ds(off, CH)], src_v)
        pltpu.sync_copy(dst.at[pl.ds(off, CH)], dst_v)
        pltpu.async_copy(table.at[src_v], rows_v, sem).wait()
        pltpu.sync_copy(rows_v, acc.at[dst_v], add=True)
        return carry

    lax.fori_loop(0, STEPS, step, 0)

    hbase = s * EPH

    def hstep(i, carry):
        off = pl.multiple_of(hbase + i * CH, 8)
        pltpu.sync_copy(dst.at[pl.ds(off, CH)], dst2_v)
        pltpu.sync_copy(ones_v, dacc.at[dst2_v], add=True)
        return carry

    lax.fori_loop(0, HSTEPS, hstep, 0)
    plsc.subcore_barrier()

    def finish(o, sz):
        pltpu.sync_copy(dacc.at[pl.ds(pl.multiple_of(o, 8), sz)],
                        dbuf.at[pl.ds(0, sz)])
        for q in range(sz // CH + (1 if sz % CH else 0)):
            qsz = min(CH, sz - q * CH)
            ro = pl.multiple_of(o + q * CH, 8)
            pltpu.sync_copy(acc.at[pl.ds(ro, qsz)], rows_v.at[pl.ds(0, qsz)])

            def scale(r, carry):
                dvec = dbuf[q * CH + r, :]
                inv = 1.0 / jnp.maximum(dvec, 1.0)
                for k in range(D_H // 16):
                    rows_v[r, pl.ds(k * 16, 16)] *= inv
                return carry

            lax.fori_loop(0, qsz, scale, 0)
            obase = pl.multiple_of(c * N + o + q * CH, 8)
            pltpu.sync_copy(rows_v.at[pl.ds(0, qsz)], out.at[pl.ds(obase, qsz)])

    for_stripe(finish)


def _make_sc_agg():
    mesh = plsc.VectorSubcoreMesh(core_axis_name="c", subcore_axis_name="s")
    return pl.kernel(
        _sc_agg_body,
        out_type=(jax.ShapeDtypeStruct((NC * N, D_H), jnp.float32),),
        mesh=mesh,
        compiler_params=pltpu.CompilerParams(use_tc_tiling_on_sc=False),
        scratch_types=(
            pltpu.VMEM((CH,), jnp.int32),
            pltpu.VMEM((CH,), jnp.int32),
            pltpu.VMEM((CH,), jnp.int32),
            pltpu.VMEM((CH, D_H), jnp.float32),
            pltpu.VMEM((CH, 16), jnp.float32),
            pltpu.VMEM((16, 16), jnp.float32),
            pltpu.VMEM((S_HI, 16), jnp.float32),
            pltpu.VMEM_SHARED((N, D_H), jnp.float32),
            pltpu.VMEM_SHARED((N, 16), jnp.float32),
            pltpu.SemaphoreType.DMA,
        ),
    )


def _sage1_body(p0, p1, x, wl, wr, b, out):
    mean = p0[...] + p1[...]
    h = jnp.dot(mean, wl[...], preferred_element_type=jnp.float32)
    h += jnp.dot(x[...], wr[...], preferred_element_type=jnp.float32)
    out[...] = jnp.maximum(h + b[...], 0.0)


def _sage2_body(p0, p1, h, wl, wr, b, wdec, bdec, eps,
                mu_o, lv_o, z_o, xr_o):
    mean = p0[...] + p1[...]
    t = jnp.dot(mean, wl[...], preferred_element_type=jnp.float32)
    t += jnp.dot(h[...], wr[...], preferred_element_type=jnp.float32)
    t += b[...]
    mu = t[:, :D_Z]
    lv = t[:, D_Z:]
    z = mu + eps[...] * jnp.exp(0.5 * lv)
    mu_o[...] = mu
    lv_o[...] = lv
    z_o[...] = z
    xr_o[...] = jnp.dot(z, wdec[...], preferred_element_type=jnp.float32) + bdec[...]


def _zzt_body(zr, zc, out):
    out[...] = lax.dot_general(
        zr[...], zc[...], (((1,), (1,)), ((), ())),
        preferred_element_type=jnp.float32)


BM = 400
GB = N // BM
BA = 400
GA = N // BA


def kernel(x, edge_index, W1_l, W1_r, b1, Wmu_l, Wmu_r, bmu,
           Wlv_l, Wlv_r, blv, Wdec, bdec):
    src = edge_index[0]
    dst = edge_index[1]
    z128 = jnp.zeros((N, D_H), jnp.float32)

    sc_agg = _make_sc_agg()
    (agg1,) = sc_agg(x, src, dst, z128)

    part_spec = pl.BlockSpec((BM, D_H), lambda i: (i, 0))
    part_spec_hi = pl.BlockSpec((BM, D_H), lambda i: (i + GB, 0))
    w_spec = pl.BlockSpec((D_H, D_H), lambda i: (0, 0))
    b_spec = pl.BlockSpec((1, D_H), lambda i: (0, 0))

    h = pl.pallas_call(
        _sage1_body,
        grid=(GB,),
        in_specs=[part_spec, part_spec_hi,
                  pl.BlockSpec((BM, D_IN), lambda i: (i, 0)),
                  w_spec, w_spec, b_spec],
        out_specs=pl.BlockSpec((BM, D_H), lambda i: (i, 0)),
        out_shape=jax.ShapeDtypeStruct((N, D_H), jnp.float32),
    )(agg1, agg1, x, W1_l, W1_r, b1.reshape(1, D_H))

    (agg2,) = sc_agg(h, src, dst, z128)

    wl_cat = jnp.concatenate([Wmu_l, Wlv_l], axis=1)
    wr_cat = jnp.concatenate([Wmu_r, Wlv_r], axis=1)
    b_cat = jnp.concatenate([bmu, blv]).reshape(1, 2 * D_Z)
    eps = jax.random.normal(jax.random.key(42), (N, D_Z), dtype=jnp.float32)

    z_out = jax.ShapeDtypeStruct((N, D_Z), jnp.float32)
    mu, logvar, z, x_recon = pl.pallas_call(
        _sage2_body,
        grid=(GB,),
        in_specs=[part_spec, part_spec_hi,
                  pl.BlockSpec((BM, D_H), lambda i: (i, 0)),
                  w_spec, w_spec, pl.BlockSpec((1, 2 * D_Z), lambda i: (0, 0)),
                  pl.BlockSpec((D_Z, D_IN), lambda i: (0, 0)),
                  pl.BlockSpec((1, D_IN), lambda i: (0, 0)),
                  pl.BlockSpec((BM, D_Z), lambda i: (i, 0))],
        out_specs=[pl.BlockSpec((BM, D_Z), lambda i: (i, 0)),
                   pl.BlockSpec((BM, D_Z), lambda i: (i, 0)),
                   pl.BlockSpec((BM, D_Z), lambda i: (i, 0)),
                   pl.BlockSpec((BM, D_IN), lambda i: (i, 0))],
        out_shape=[z_out, z_out, z_out,
                   jax.ShapeDtypeStruct((N, D_IN), jnp.float32)],
    )(agg2, agg2, h, wl_cat, wr_cat, b_cat,
      Wdec, bdec.reshape(1, D_IN), eps)

    adj = pl.pallas_call(
        _zzt_body,
        grid=(GA,),
        in_specs=[pl.BlockSpec((BA, D_Z), lambda i: (i, 0)),
                  pl.BlockSpec((N, D_Z), lambda i: (0, 0))],
        out_specs=pl.BlockSpec((BA, N), lambda i: (i, 0)),
        out_shape=jax.ShapeDtypeStruct((N, N), jnp.float32),
    )(z, z)

    return (x_recon, adj, mu, logvar)


# speedup vs baseline: 7.5102x; 2.1002x over previous
import functools

import jax
import jax.numpy as jnp
from jax import lax
from jax.experimental import pallas as pl
from jax.experimental.pallas import tpu as pltpu
from jax.experimental.pallas import tpu_sc as plsc

N = 10000
E = 320000
D_IN = 128
D_H = 128
D_Z = 64

NC = 2
NS = 16
CH = 80
EPW = E // (NC * NS)
STEPS = EPW // CH
EPH = E // NS
HSTEPS = EPH // CH
S_LO = 624
S_HI = N - S_LO * (NS - 1)


BF = 25
MBLK = STEPS // BF
HBLK = HSTEPS // BF


def _sc_agg_body(with_deg, *refs):
    if with_deg:
        (table, src2d, dst2d, z128, out, dout,
         src_m, dst_m, rows0, rows1, ones_v, zbuf, dbuf, acc, dacc,
         sem0, sem1) = refs
    else:
        (table, src2d, dst2d, z128, deg_in, out,
         src_m, dst_m, rows0, rows1, dbuf, acc,
         sem0, sem1) = refs
    bufs = (rows0, rows1)
    sems = (sem0, sem1)

    c = lax.axis_index("c")
    s = lax.axis_index("s")

    def for_stripe(fn):
        @pl.when(s < NS - 1)
        def _lo():
            fn(pl.multiple_of(s * S_LO, 8), S_LO)

        @pl.when(s == NS - 1)
        def _hi():
            fn((NS - 1) * S_LO, S_HI)

    if with_deg:
        zeros16 = jnp.zeros((16,), jnp.float32)
        ones16 = jnp.ones((16,), jnp.float32)
        for j in range(16):
            zbuf[j, :] = zeros16
        for j in range(CH):
            ones_v[j, :] = ones16

    def zero(o, sz):
        pltpu.sync_copy(z128.at[pl.ds(o, sz)], acc.at[pl.ds(o, sz)])
        if with_deg:
            for k in range(sz // 16):
                pltpu.sync_copy(zbuf, dacc.at[pl.ds(pl.multiple_of(o + k * 16, 8), 16)])

    for_stripe(zero)
    plsc.subcore_barrier()

    mbase = (c * NS + s) * STEPS

    def mblk(b, carry):
        row0 = mbase + b * BF
        pltpu.sync_copy(src2d.at[pl.ds(row0, BF)], src_m)
        pltpu.sync_copy(dst2d.at[pl.ds(row0, BF)], dst_m)
        pend = None
        for j in range(BF):
            h = pltpu.async_copy(table.at[src_m.at[j]], bufs[j % 2], sems[j % 2])
            if pend is not None:
                pend.wait()
                pltpu.sync_copy(bufs[(j - 1) % 2], acc.at[dst_m.at[j - 1]], add=True)
            pend = h
        pend.wait()
        pltpu.sync_copy(bufs[(BF - 1) % 2], acc.at[dst_m.at[BF - 1]], add=True)
        return carry

    lax.fori_loop(0, MBLK, mblk, 0)

    if with_deg:
        hbase = s * (HBLK * BF)

        def hblk(b, carry):
            row0 = hbase + b * BF
            pltpu.sync_copy(dst2d.at[pl.ds(row0, BF)], dst_m)
            for j in range(BF):
                pltpu.sync_copy(ones_v, dacc.at[dst_m.at[j]], add=True)
            return carry

        lax.fori_loop(0, HBLK, hblk, 0)
    plsc.subcore_barrier()

    def finish(o, sz):
        if with_deg:
            pltpu.sync_copy(dacc.at[pl.ds(pl.multiple_of(o, 8), sz)],
                            dbuf.at[pl.ds(0, sz)])

            @pl.when(c == 0)
            def _export():
                pltpu.sync_copy(dacc.at[pl.ds(pl.multiple_of(o, 8), sz)],
                                dout.at[pl.ds(pl.multiple_of(o, 8), sz)])
        else:
            pltpu.sync_copy(deg_in.at[pl.ds(pl.multiple_of(o, 8), sz)],
                            dbuf.at[pl.ds(0, sz)])
        for q in range(sz // CH + (1 if sz % CH else 0)):
            qsz = min(CH, sz - q * CH)
            ro = pl.multiple_of(o + q * CH, 8)
            pltpu.sync_copy(acc.at[pl.ds(ro, qsz)], rows0.at[pl.ds(0, qsz)])

            def scale(r, carry):
                dvec = dbuf[q * CH + r, :]
                inv = 1.0 / jnp.maximum(dvec, 1.0)
                for k in range(D_H // 16):
                    rows0[r, pl.ds(k * 16, 16)] *= inv
                return carry

            lax.fori_loop(0, qsz, scale, 0)
            obase = pl.multiple_of(c * N + o + q * CH, 8)
            pltpu.sync_copy(rows0.at[pl.ds(0, qsz)], out.at[pl.ds(obase, qsz)])

    for_stripe(finish)


def _make_sc_agg(with_deg):
    mesh = plsc.VectorSubcoreMesh(core_axis_name="c", subcore_axis_name="s")
    out_type = [jax.ShapeDtypeStruct((NC * N, D_H), jnp.float32)]
    if with_deg:
        out_type.append(jax.ShapeDtypeStruct((N, 16), jnp.float32))
    scratch = [
        pltpu.VMEM((BF, CH), jnp.int32),
        pltpu.VMEM((BF, CH), jnp.int32),
        pltpu.VMEM((CH, D_H), jnp.float32),
        pltpu.VMEM((CH, D_H), jnp.float32),
    ]
    if with_deg:
        scratch += [
            pltpu.VMEM((CH, 16), jnp.float32),
            pltpu.VMEM((16, 16), jnp.float32),
        ]
    scratch += [
        pltpu.VMEM((S_HI, 16), jnp.float32),
        pltpu.VMEM_SHARED((N, D_H), jnp.float32),
    ]
    if with_deg:
        scratch.append(pltpu.VMEM_SHARED((N, 16), jnp.float32))
    scratch += [pltpu.SemaphoreType.DMA, pltpu.SemaphoreType.DMA]
    return pl.kernel(
        functools.partial(_sc_agg_body, with_deg),
        out_type=tuple(out_type),
        mesh=mesh,
        compiler_params=pltpu.CompilerParams(use_tc_tiling_on_sc=False),
        scratch_types=tuple(scratch),
    )


def _sage1_body(p0, p1, x, wl, wr, b, out):
    mean = p0[...] + p1[...]
    h = jnp.dot(mean, wl[...], preferred_element_type=jnp.float32)
    h += jnp.dot(x[...], wr[...], preferred_element_type=jnp.float32)
    out[...] = jnp.maximum(h + b[...], 0.0)


def _sage2_body(p0, p1, h, wl, wr, b, wdec, bdec, eps,
                mu_o, lv_o, z_o, xr_o):
    mean = p0[...] + p1[...]
    t = jnp.dot(mean, wl[...], preferred_element_type=jnp.float32)
    t += jnp.dot(h[...], wr[...], preferred_element_type=jnp.float32)
    t += b[...]
    mu = t[:, :D_Z]
    lv = t[:, D_Z:]
    z = mu + eps[...] * jnp.exp(0.5 * lv)
    mu_o[...] = mu
    lv_o[...] = lv
    z_o[...] = z
    xr_o[...] = jnp.dot(z, wdec[...], preferred_element_type=jnp.float32) + bdec[...]


def _zzt_body(zr, zc, out):
    out[...] = lax.dot_general(
        zr[...], zc[...], (((1,), (1,)), ((), ())),
        preferred_element_type=jnp.float32)


BM = 400
GB = N // BM
BA = 400
GA = N // BA


def kernel(x, edge_index, W1_l, W1_r, b1, Wmu_l, Wmu_r, bmu,
           Wlv_l, Wlv_r, blv, Wdec, bdec):
    src2d = edge_index[0].reshape(E // CH, CH)
    dst2d = edge_index[1].reshape(E // CH, CH)
    z128 = jnp.zeros((N, D_H), jnp.float32)

    agg1, deg = _make_sc_agg(True)(x, src2d, dst2d, z128)

    part_spec = pl.BlockSpec((BM, D_H), lambda i: (i, 0))
    part_spec_hi = pl.BlockSpec((BM, D_H), lambda i: (i + GB, 0))
    w_spec = pl.BlockSpec((D_H, D_H), lambda i: (0, 0))
    b_spec = pl.BlockSpec((1, D_H), lambda i: (0, 0))

    h = pl.pallas_call(
        _sage1_body,
        grid=(GB,),
        in_specs=[part_spec, part_spec_hi,
                  pl.BlockSpec((BM, D_IN), lambda i: (i, 0)),
                  w_spec, w_spec, b_spec],
        out_specs=pl.BlockSpec((BM, D_H), lambda i: (i, 0)),
        out_shape=jax.ShapeDtypeStruct((N, D_H), jnp.float32),
    )(agg1, agg1, x, W1_l, W1_r, b1.reshape(1, D_H))

    (agg2,) = _make_sc_agg(False)(h, src2d, dst2d, z128, deg)

    wl_cat = jnp.concatenate([Wmu_l, Wlv_l], axis=1)
    wr_cat = jnp.concatenate([Wmu_r, Wlv_r], axis=1)
    b_cat = jnp.concatenate([bmu, blv]).reshape(1, 2 * D_Z)
    eps = jax.random.normal(jax.random.key(42), (N, D_Z), dtype=jnp.float32)

    z_out = jax.ShapeDtypeStruct((N, D_Z), jnp.float32)
    mu, logvar, z, x_recon = pl.pallas_call(
        _sage2_body,
        grid=(GB,),
        in_specs=[part_spec, part_spec_hi,
                  pl.BlockSpec((BM, D_H), lambda i: (i, 0)),
                  w_spec, w_spec, pl.BlockSpec((1, 2 * D_Z), lambda i: (0, 0)),
                  pl.BlockSpec((D_Z, D_IN), lambda i: (0, 0)),
                  pl.BlockSpec((1, D_IN), lambda i: (0, 0)),
                  pl.BlockSpec((BM, D_Z), lambda i: (i, 0))],
        out_specs=[pl.BlockSpec((BM, D_Z), lambda i: (i, 0)),
                   pl.BlockSpec((BM, D_Z), lambda i: (i, 0)),
                   pl.BlockSpec((BM, D_Z), lambda i: (i, 0)),
                   pl.BlockSpec((BM, D_IN), lambda i: (i, 0))],
        out_shape=[z_out, z_out, z_out,
                   jax.ShapeDtypeStruct((N, D_IN), jnp.float32)],
    )(agg2, agg2, h, wl_cat, wr_cat, b_cat,
      Wdec, bdec.reshape(1, D_IN), eps)

    adj = pl.pallas_call(
        _zzt_body,
        grid=(GA,),
        in_specs=[pl.BlockSpec((BA, D_Z), lambda i: (i, 0)),
                  pl.BlockSpec((N, D_Z), lambda i: (0, 0))],
        out_specs=pl.BlockSpec((BA, N), lambda i: (i, 0)),
        out_shape=jax.ShapeDtypeStruct((N, N), jnp.float32),
    )(z, z)

    return (x_recon, adj, mu, logvar)


# speedup vs baseline: 7.7946x; 1.0379x over previous
import functools

import jax
import jax.numpy as jnp
from jax import lax
from jax.experimental import pallas as pl
from jax.experimental.pallas import tpu as pltpu
from jax.experimental.pallas import tpu_sc as plsc

N = 10000
E = 320000
D_IN = 128
D_H = 128
D_Z = 64

NC = 2
NS = 16
CH = 80
EPW = E // (NC * NS)
STEPS = EPW // CH
EPH = E // NS
HSTEPS = EPH // CH
S_LO = 624
S_HI = N - S_LO * (NS - 1)


BF = 25
MBLK = STEPS // BF
HBLK = HSTEPS // BF


def _sc_agg_body(with_deg, *refs):
    if with_deg:
        (table, src2d, dst2d, z128, out, dout,
         src_m, dst_m, dsth_m, rows0, rows1, ones_v, zbuf, dbuf, acc, dacc,
         sem0, sem1) = refs
    else:
        (table, src2d, dst2d, z128, deg_in, out,
         src_m, dst_m, rows0, rows1, dbuf, acc,
         sem0, sem1) = refs
    bufs = (rows0, rows1)
    sems = (sem0, sem1)

    c = lax.axis_index("c")
    s = lax.axis_index("s")

    def for_stripe(fn):
        @pl.when(s < NS - 1)
        def _lo():
            fn(pl.multiple_of(s * S_LO, 8), S_LO)

        @pl.when(s == NS - 1)
        def _hi():
            fn((NS - 1) * S_LO, S_HI)

    if with_deg:
        zeros16 = jnp.zeros((16,), jnp.float32)
        ones16 = jnp.ones((16,), jnp.float32)
        for j in range(16):
            zbuf[j, :] = zeros16
        for j in range(CH):
            ones_v[j, :] = ones16

    def zero(o, sz):
        pltpu.sync_copy(z128.at[pl.ds(o, sz)], acc.at[pl.ds(o, sz)])
        if with_deg:
            for k in range(sz // 16):
                pltpu.sync_copy(zbuf, dacc.at[pl.ds(pl.multiple_of(o + k * 16, 8), 16)])

    for_stripe(zero)
    plsc.subcore_barrier()

    mbase = (c * NS + s) * STEPS

    def scatter_step(j):
        pltpu.sync_copy(bufs[j % 2], acc.at[dst_m.at[j]], add=True)
        if with_deg:
            pltpu.sync_copy(ones_v, dacc.at[dsth_m.at[2 * j]], add=True)
            pltpu.sync_copy(ones_v, dacc.at[dsth_m.at[2 * j + 1]], add=True)

    def mblk(b, carry):
        row0 = mbase + b * BF
        pltpu.sync_copy(src2d.at[pl.ds(row0, BF)], src_m)
        pltpu.sync_copy(dst2d.at[pl.ds(row0, BF)], dst_m)
        if with_deg:
            pltpu.sync_copy(dst2d.at[pl.ds(s * HSTEPS + b * 2 * BF, 2 * BF)], dsth_m)
        pend = None
        for j in range(BF):
            h = pltpu.async_copy(table.at[src_m.at[j]], bufs[j % 2], sems[j % 2])
            if pend is not None:
                pend.wait()
                scatter_step(j - 1)
            pend = h
        pend.wait()
        scatter_step(BF - 1)
        return carry

    lax.fori_loop(0, MBLK, mblk, 0)
    plsc.subcore_barrier()

    def finish(o, sz):
        if with_deg:
            pltpu.sync_copy(dacc.at[pl.ds(pl.multiple_of(o, 8), sz)],
                            dbuf.at[pl.ds(0, sz)])

            @pl.when(c == 0)
            def _export():
                pltpu.sync_copy(dacc.at[pl.ds(pl.multiple_of(o, 8), sz)],
                                dout.at[pl.ds(pl.multiple_of(o, 8), sz)])
        else:
            pltpu.sync_copy(deg_in.at[pl.ds(pl.multiple_of(o, 8), sz)],
                            dbuf.at[pl.ds(0, sz)])
        for q in range(sz // CH + (1 if sz % CH else 0)):
            qsz = min(CH, sz - q * CH)
            ro = pl.multiple_of(o + q * CH, 8)
            pltpu.sync_copy(acc.at[pl.ds(ro, qsz)], rows0.at[pl.ds(0, qsz)])

            def scale(r, carry):
                dvec = dbuf[q * CH + r, :]
                inv = 1.0 / jnp.maximum(dvec, 1.0)
                for k in range(D_H // 16):
                    rows0[r, pl.ds(k * 16, 16)] *= inv
                return carry

            lax.fori_loop(0, qsz, scale, 0)
            obase = pl.multiple_of(c * N + o + q * CH, 8)
            pltpu.sync_copy(rows0.at[pl.ds(0, qsz)], out.at[pl.ds(obase, qsz)])

    for_stripe(finish)


def _make_sc_agg(with_deg):
    mesh = plsc.VectorSubcoreMesh(core_axis_name="c", subcore_axis_name="s")
    out_type = [jax.ShapeDtypeStruct((NC * N, D_H), jnp.float32)]
    if with_deg:
        out_type.append(jax.ShapeDtypeStruct((N, 16), jnp.float32))
    scratch = [
        pltpu.VMEM((BF, CH), jnp.int32),
        pltpu.VMEM((BF, CH), jnp.int32),
    ]
    if with_deg:
        scratch.append(pltpu.VMEM((2 * BF, CH), jnp.int32))
    scratch += [
        pltpu.VMEM((CH, D_H), jnp.float32),
        pltpu.VMEM((CH, D_H), jnp.float32),
    ]
    if with_deg:
        scratch += [
            pltpu.VMEM((CH, 16), jnp.float32),
            pltpu.VMEM((16, 16), jnp.float32),
        ]
    scratch += [
        pltpu.VMEM((S_HI, 16), jnp.float32),
        pltpu.VMEM_SHARED((N, D_H), jnp.float32),
    ]
    if with_deg:
        scratch.append(pltpu.VMEM_SHARED((N, 16), jnp.float32))
    scratch += [pltpu.SemaphoreType.DMA, pltpu.SemaphoreType.DMA]
    return pl.kernel(
        functools.partial(_sc_agg_body, with_deg),
        out_type=tuple(out_type),
        mesh=mesh,
        compiler_params=pltpu.CompilerParams(use_tc_tiling_on_sc=False),
        scratch_types=tuple(scratch),
    )


def _sage1_body(p0, p1, x, wl, wr, b, out):
    mean = p0[...] + p1[...]
    h = jnp.dot(mean, wl[...], preferred_element_type=jnp.float32)
    h += jnp.dot(x[...], wr[...], preferred_element_type=jnp.float32)
    out[...] = jnp.maximum(h + b[...], 0.0)


def _sage2_body(p0, p1, h, wl, wr, b, wdec, bdec, eps,
                mu_o, lv_o, z_o, xr_o):
    mean = p0[...] + p1[...]
    t = jnp.dot(mean, wl[...], preferred_element_type=jnp.float32)
    t += jnp.dot(h[...], wr[...], preferred_element_type=jnp.float32)
    t += b[...]
    mu = t[:, :D_Z]
    lv = t[:, D_Z:]
    z = mu + eps[...] * jnp.exp(0.5 * lv)
    mu_o[...] = mu
    lv_o[...] = lv
    z_o[...] = z
    xr_o[...] = jnp.dot(z, wdec[...], preferred_element_type=jnp.float32) + bdec[...]


def _zzt_body(zr, zc, out):
    out[...] = lax.dot_general(
        zr[...], zc[...], (((1,), (1,)), ((), ())),
        preferred_element_type=jnp.float32)


BM = 400
GB = N // BM
BA = 400
GA = N // BA


def kernel(x, edge_index, W1_l, W1_r, b1, Wmu_l, Wmu_r, bmu,
           Wlv_l, Wlv_r, blv, Wdec, bdec):
    src2d = edge_index[0].reshape(E // CH, CH)
    dst2d = edge_index[1].reshape(E // CH, CH)
    z128 = jnp.zeros((N, D_H), jnp.float32)

    agg1, deg = _make_sc_agg(True)(x, src2d, dst2d, z128)

    part_spec = pl.BlockSpec((BM, D_H), lambda i: (i, 0))
    part_spec_hi = pl.BlockSpec((BM, D_H), lambda i: (i + GB, 0))
    w_spec = pl.BlockSpec((D_H, D_H), lambda i: (0, 0))
    b_spec = pl.BlockSpec((1, D_H), lambda i: (0, 0))

    h = pl.pallas_call(
        _sage1_body,
        grid=(GB,),
        in_specs=[part_spec, part_spec_hi,
                  pl.BlockSpec((BM, D_IN), lambda i: (i, 0)),
                  w_spec, w_spec, b_spec],
        out_specs=pl.BlockSpec((BM, D_H), lambda i: (i, 0)),
        out_shape=jax.ShapeDtypeStruct((N, D_H), jnp.float32),
    )(agg1, agg1, x, W1_l, W1_r, b1.reshape(1, D_H))

    (agg2,) = _make_sc_agg(False)(h, src2d, dst2d, z128, deg)

    wl_cat = jnp.concatenate([Wmu_l, Wlv_l], axis=1)
    wr_cat = jnp.concatenate([Wmu_r, Wlv_r], axis=1)
    b_cat = jnp.concatenate([bmu, blv]).reshape(1, 2 * D_Z)
    eps = jax.random.normal(jax.random.key(42), (N, D_Z), dtype=jnp.float32)

    z_out = jax.ShapeDtypeStruct((N, D_Z), jnp.float32)
    mu, logvar, z, x_recon = pl.pallas_call(
        _sage2_body,
        grid=(GB,),
        in_specs=[part_spec, part_spec_hi,
                  pl.BlockSpec((BM, D_H), lambda i: (i, 0)),
                  w_spec, w_spec, pl.BlockSpec((1, 2 * D_Z), lambda i: (0, 0)),
                  pl.BlockSpec((D_Z, D_IN), lambda i: (0, 0)),
                  pl.BlockSpec((1, D_IN), lambda i: (0, 0)),
                  pl.BlockSpec((BM, D_Z), lambda i: (i, 0))],
        out_specs=[pl.BlockSpec((BM, D_Z), lambda i: (i, 0)),
                   pl.BlockSpec((BM, D_Z), lambda i: (i, 0)),
                   pl.BlockSpec((BM, D_Z), lambda i: (i, 0)),
                   pl.BlockSpec((BM, D_IN), lambda i: (i, 0))],
        out_shape=[z_out, z_out, z_out,
                   jax.ShapeDtypeStruct((N, D_IN), jnp.float32)],
    )(agg2, agg2, h, wl_cat, wr_cat, b_cat,
      Wdec, bdec.reshape(1, D_IN), eps)

    adj = pl.pallas_call(
        _zzt_body,
        grid=(GA,),
        in_specs=[pl.BlockSpec((BA, D_Z), lambda i: (i, 0)),
                  pl.BlockSpec((N, D_Z), lambda i: (0, 0))],
        out_specs=pl.BlockSpec((BA, N), lambda i: (i, 0)),
        out_shape=jax.ShapeDtypeStruct((N, N), jnp.float32),
    )(z, z)

    return (x_recon, adj, mu, logvar)


# speedup vs baseline: 8.3000x; 1.0648x over previous
import functools

import jax
import jax.numpy as jnp
from jax import lax
from jax.experimental import pallas as pl
from jax.experimental.pallas import tpu as pltpu
from jax.experimental.pallas import tpu_sc as plsc

N = 10000
E = 320000
D_IN = 128
D_H = 128
D_Z = 64

NC = 2
NS = 16
CH = 80
EPW = E // (NC * NS)
STEPS = EPW // CH
EPH = E // NS
HSTEPS = EPH // CH
S_LO = 624
S_HI = N - S_LO * (NS - 1)


BF = 25
MBLK = STEPS // BF
HBLK = HSTEPS // BF


def _sc_agg_body(with_deg, *refs):
    if with_deg:
        (table, src2d, dst2d, z128, out, dout,
         src_m, dst_m, dsth_m, rows0, rows1, ones_v, zbuf, dbuf, acc, dacc,
         gsem0, gsem1, ssem0, ssem1, osem0, osem1) = refs
        bufs = (rows0, rows1)
        gsems = (gsem0, gsem1)
        ssems = (ssem0, ssem1)
    else:
        (table, src2d, dst2d, z128, deg_in, out,
         src_m, dst_m, rows0, rows1, rows2, dbuf, acc,
         gsem0, gsem1, gsem2, ssem0, ssem1, ssem2, osem0, osem1) = refs
        bufs = (rows0, rows1, rows2)
        gsems = (gsem0, gsem1, gsem2)
        ssems = (ssem0, ssem1, ssem2)
    osems = (osem0, osem1)
    NB = len(bufs)

    c = lax.axis_index("c")
    s = lax.axis_index("s")

    def for_stripe(fn):
        @pl.when(s < NS - 1)
        def _lo():
            fn(pl.multiple_of(s * S_LO, 8), S_LO)

        @pl.when(s == NS - 1)
        def _hi():
            fn((NS - 1) * S_LO, S_HI)

    if with_deg:
        zeros16 = jnp.zeros((16,), jnp.float32)
        ones16 = jnp.ones((16,), jnp.float32)
        for j in range(16):
            zbuf[j, :] = zeros16
        for j in range(CH):
            ones_v[j, :] = ones16

    def zero(o, sz):
        pltpu.sync_copy(z128.at[pl.ds(o, sz)], acc.at[pl.ds(o, sz)])
        if with_deg:
            for k in range(sz // 16):
                pltpu.sync_copy(zbuf, dacc.at[pl.ds(pl.multiple_of(o + k * 16, 8), 16)])

    for_stripe(zero)
    plsc.subcore_barrier()

    mbase = (c * NS + s) * STEPS

    def scatter_step(j):
        h = pltpu.async_copy(bufs[j % NB], acc.at[dst_m.at[j]],
                             ssems[j % NB], add=True)
        if with_deg:
            pltpu.sync_copy(ones_v, dacc.at[dsth_m.at[2 * j]], add=True)
            pltpu.sync_copy(ones_v, dacc.at[dsth_m.at[2 * j + 1]], add=True)
        return h

    def mblk(b, carry):
        row0 = mbase + b * BF
        pltpu.sync_copy(src2d.at[pl.ds(row0, BF)], src_m)
        pltpu.sync_copy(dst2d.at[pl.ds(row0, BF)], dst_m)
        if with_deg:
            pltpu.sync_copy(dst2d.at[pl.ds(s * HSTEPS + b * 2 * BF, 2 * BF)], dsth_m)
        gh, sh = {}, {}
        for j in range(BF):
            if j >= NB:
                sh[j - NB].wait()
            gh[j] = pltpu.async_copy(table.at[src_m.at[j]], bufs[j % NB],
                                     gsems[j % NB])
            if j >= 1:
                gh[j - 1].wait()
                sh[j - 1] = scatter_step(j - 1)
        gh[BF - 1].wait()
        sh[BF - 1] = scatter_step(BF - 1)
        for j in range(BF - NB, BF):
            sh[j].wait()
        return carry

    lax.fori_loop(0, MBLK, mblk, 0)
    plsc.subcore_barrier()

    def finish(o, sz):
        if with_deg:
            pltpu.sync_copy(dacc.at[pl.ds(pl.multiple_of(o, 8), sz)],
                            dbuf.at[pl.ds(0, sz)])

            @pl.when(c == 0)
            def _export():
                pltpu.sync_copy(dacc.at[pl.ds(pl.multiple_of(o, 8), sz)],
                                dout.at[pl.ds(pl.multiple_of(o, 8), sz)])
        else:
            pltpu.sync_copy(deg_in.at[pl.ds(pl.multiple_of(o, 8), sz)],
                            dbuf.at[pl.ds(0, sz)])
        nq = sz // CH + (1 if sz % CH else 0)
        qsizes = [min(CH, sz - q * CH) for q in range(nq)]
        hin, hout = {}, {}

        def issue_in(q):
            ro = pl.multiple_of(o + q * CH, 8)
            hin[q] = pltpu.async_copy(acc.at[pl.ds(ro, qsizes[q])],
                                      bufs[q % 2].at[pl.ds(0, qsizes[q])],
                                      gsems[q % 2])

        issue_in(0)
        for q in range(nq):
            if q + 1 < nq:
                if q >= 1:
                    hout[q - 1].wait()
                issue_in(q + 1)
            hin[q].wait()
            buf = bufs[q % 2]

            def scale(r, carry):
                dvec = dbuf[q * CH + r, :]
                inv = 1.0 / jnp.maximum(dvec, 1.0)
                for k in range(D_H // 16):
                    buf[r, pl.ds(k * 16, 16)] *= inv
                return carry

            lax.fori_loop(0, qsizes[q], scale, 0)
            obase = pl.multiple_of(c * N + o + q * CH, 8)
            hout[q] = pltpu.async_copy(buf.at[pl.ds(0, qsizes[q])],
                                       out.at[pl.ds(obase, qsizes[q])],
                                       osems[q % 2])
        for q in range(max(0, nq - 2), nq):
            hout[q].wait()

    for_stripe(finish)


def _make_sc_agg(with_deg):
    mesh = plsc.VectorSubcoreMesh(core_axis_name="c", subcore_axis_name="s")
    out_type = [jax.ShapeDtypeStruct((NC * N, D_H), jnp.float32)]
    if with_deg:
        out_type.append(jax.ShapeDtypeStruct((N, 16), jnp.float32))
    scratch = [
        pltpu.VMEM((BF, CH), jnp.int32),
        pltpu.VMEM((BF, CH), jnp.int32),
    ]
    if with_deg:
        scratch.append(pltpu.VMEM((2 * BF, CH), jnp.int32))
    scratch += [
        pltpu.VMEM((CH, D_H), jnp.float32),
        pltpu.VMEM((CH, D_H), jnp.float32),
    ]
    if not with_deg:
        scratch.append(pltpu.VMEM((CH, D_H), jnp.float32))
    if with_deg:
        scratch += [
            pltpu.VMEM((CH, 16), jnp.float32),
            pltpu.VMEM((16, 16), jnp.float32),
        ]
    scratch += [
        pltpu.VMEM((S_HI, 16), jnp.float32),
        pltpu.VMEM_SHARED((N, D_H), jnp.float32),
    ]
    if with_deg:
        scratch.append(pltpu.VMEM_SHARED((N, 16), jnp.float32))
    nsem = 6 if with_deg else 8
    scratch += [pltpu.SemaphoreType.DMA] * nsem
    return pl.kernel(
        functools.partial(_sc_agg_body, with_deg),
        out_type=tuple(out_type),
        mesh=mesh,
        compiler_params=pltpu.CompilerParams(use_tc_tiling_on_sc=False),
        scratch_types=tuple(scratch),
    )


def _sage1_body(p0, p1, x, wl, wr, b, out):
    mean = p0[...] + p1[...]
    h = jnp.dot(mean, wl[...], preferred_element_type=jnp.float32)
    h += jnp.dot(x[...], wr[...], preferred_element_type=jnp.float32)
    out[...] = jnp.maximum(h + b[...], 0.0)


def _sage2_body(p0, p1, h, wl, wr, b, wdec, bdec, eps,
                mu_o, lv_o, z_o, xr_o):
    mean = p0[...] + p1[...]
    t = jnp.dot(mean, wl[...], preferred_element_type=jnp.float32)
    t += jnp.dot(h[...], wr[...], preferred_element_type=jnp.float32)
    t += b[...]
    mu = t[:, :D_Z]
    lv = t[:, D_Z:]
    z = mu + eps[...] * jnp.exp(0.5 * lv)
    mu_o[...] = mu
    lv_o[...] = lv
    z_o[...] = z
    xr_o[...] = jnp.dot(z, wdec[...], preferred_element_type=jnp.float32) + bdec[...]


def _zzt_body(zr, zc, out):
    out[...] = lax.dot_general(
        zr[...], zc[...], (((1,), (1,)), ((), ())),
        preferred_element_type=jnp.float32)


BM = 400
GB = N // BM
BA = 400
GA = N // BA


def kernel(x, edge_index, W1_l, W1_r, b1, Wmu_l, Wmu_r, bmu,
           Wlv_l, Wlv_r, blv, Wdec, bdec):
    src2d = edge_index[0].reshape(E // CH, CH)
    dst2d = edge_index[1].reshape(E // CH, CH)
    z128 = jnp.zeros((N, D_H), jnp.float32)

    agg1, deg = _make_sc_agg(True)(x, src2d, dst2d, z128)

    part_spec = pl.BlockSpec((BM, D_H), lambda i: (i, 0))
    part_spec_hi = pl.BlockSpec((BM, D_H), lambda i: (i + GB, 0))
    w_spec = pl.BlockSpec((D_H, D_H), lambda i: (0, 0))
    b_spec = pl.BlockSpec((1, D_H), lambda i: (0, 0))

    h = pl.pallas_call(
        _sage1_body,
        grid=(GB,),
        in_specs=[part_spec, part_spec_hi,
                  pl.BlockSpec((BM, D_IN), lambda i: (i, 0)),
                  w_spec, w_spec, b_spec],
        out_specs=pl.BlockSpec((BM, D_H), lambda i: (i, 0)),
        out_shape=jax.ShapeDtypeStruct((N, D_H), jnp.float32),
    )(agg1, agg1, x, W1_l, W1_r, b1.reshape(1, D_H))

    (agg2,) = _make_sc_agg(False)(h, src2d, dst2d, z128, deg)

    wl_cat = jnp.concatenate([Wmu_l, Wlv_l], axis=1)
    wr_cat = jnp.concatenate([Wmu_r, Wlv_r], axis=1)
    b_cat = jnp.concatenate([bmu, blv]).reshape(1, 2 * D_Z)
    eps = jax.random.normal(jax.random.key(42), (N, D_Z), dtype=jnp.float32)

    z_out = jax.ShapeDtypeStruct((N, D_Z), jnp.float32)
    mu, logvar, z, x_recon = pl.pallas_call(
        _sage2_body,
        grid=(GB,),
        in_specs=[part_spec, part_spec_hi,
                  pl.BlockSpec((BM, D_H), lambda i: (i, 0)),
                  w_spec, w_spec, pl.BlockSpec((1, 2 * D_Z), lambda i: (0, 0)),
                  pl.BlockSpec((D_Z, D_IN), lambda i: (0, 0)),
                  pl.BlockSpec((1, D_IN), lambda i: (0, 0)),
                  pl.BlockSpec((BM, D_Z), lambda i: (i, 0))],
        out_specs=[pl.BlockSpec((BM, D_Z), lambda i: (i, 0)),
                   pl.BlockSpec((BM, D_Z), lambda i: (i, 0)),
                   pl.BlockSpec((BM, D_Z), lambda i: (i, 0)),
                   pl.BlockSpec((BM, D_IN), lambda i: (i, 0))],
        out_shape=[z_out, z_out, z_out,
                   jax.ShapeDtypeStruct((N, D_IN), jnp.float32)],
    )(agg2, agg2, h, wl_cat, wr_cat, b_cat,
      Wdec, bdec.reshape(1, D_IN), eps)

    adj = pl.pallas_call(
        _zzt_body,
        grid=(GA,),
        in_specs=[pl.BlockSpec((BA, D_Z), lambda i: (i, 0)),
                  pl.BlockSpec((N, D_Z), lambda i: (0, 0))],
        out_specs=pl.BlockSpec((BA, N), lambda i: (i, 0)),
        out_shape=jax.ShapeDtypeStruct((N, N), jnp.float32),
    )(z, z)

    return (x_recon, adj, mu, logvar)


# speedup vs baseline: 8.3008x; 1.0001x over previous
import functools

import jax
import jax.numpy as jnp
from jax import lax
from jax.experimental import pallas as pl
from jax.experimental.pallas import tpu as pltpu
from jax.experimental.pallas import tpu_sc as plsc

N = 10000
E = 320000
D_IN = 128
D_H = 128
D_Z = 64

NC = 2
NS = 16
CH = 80
EPW = E // (NC * NS)
STEPS = EPW // CH
EPH = E // NS
HSTEPS = EPH // CH
S_LO = 624
S_HI = N - S_LO * (NS - 1)


BF = 25
MBLK = STEPS // BF
HBLK = HSTEPS // BF


def _sc_agg_body(with_deg, *refs):
    if with_deg:
        (table, src2d, dst2d, z128, out, dout,
         src_m, dst_m, dsth_m, rows0, rows1, ones_v, dbuf, acc, dacc,
         gsem0, gsem1, ssem0, ssem1, osem0, osem1) = refs
        bufs = (rows0, rows1)
        gsems = (gsem0, gsem1)
        ssems = (ssem0, ssem1)
    else:
        (table, src2d, dst2d, z128, deg_in, out,
         src_m, dst_m, rows0, rows1, rows2, dbuf, acc,
         gsem0, gsem1, gsem2, ssem0, ssem1, ssem2, osem0, osem1) = refs
        bufs = (rows0, rows1, rows2)
        gsems = (gsem0, gsem1, gsem2)
        ssems = (ssem0, ssem1, ssem2)
    osems = (osem0, osem1)
    NB = len(bufs)

    c = lax.axis_index("c")
    s = lax.axis_index("s")

    def for_stripe(fn):
        @pl.when(s < NS - 1)
        def _lo():
            fn(pl.multiple_of(s * S_LO, 8), S_LO)

        @pl.when(s == NS - 1)
        def _hi():
            fn((NS - 1) * S_LO, S_HI)

    if with_deg:
        zeros16 = jnp.zeros((16,), jnp.float32)
        ones16 = jnp.ones((16,), jnp.float32)
        for j in range(CH):
            ones_v[j, :] = ones16

        def zrow(r, carry):
            dbuf[r, :] = zeros16
            return carry

        lax.fori_loop(0, S_HI, zrow, 0)

    def zero(o, sz):
        pltpu.sync_copy(z128.at[pl.ds(o, sz)], acc.at[pl.ds(o, sz)])
        if with_deg:
            pltpu.sync_copy(dbuf.at[pl.ds(0, sz)],
                            dacc.at[pl.ds(pl.multiple_of(o, 8), sz)])

    for_stripe(zero)
    plsc.subcore_barrier()

    mbase = (c * NS + s) * STEPS

    def scatter_step(j):
        h = pltpu.async_copy(bufs[j % NB], acc.at[dst_m.at[j]],
                             ssems[j % NB], add=True)
        if with_deg:
            pltpu.sync_copy(ones_v, dacc.at[dsth_m.at[2 * j]], add=True)
            pltpu.sync_copy(ones_v, dacc.at[dsth_m.at[2 * j + 1]], add=True)
        return h

    def mblk(b, carry):
        row0 = mbase + b * BF
        pltpu.sync_copy(src2d.at[pl.ds(row0, BF)], src_m)
        pltpu.sync_copy(dst2d.at[pl.ds(row0, BF)], dst_m)
        if with_deg:
            pltpu.sync_copy(dst2d.at[pl.ds(s * HSTEPS + b * 2 * BF, 2 * BF)], dsth_m)
        gh, sh = {}, {}
        for j in range(BF):
            if j >= NB:
                sh[j - NB].wait()
            gh[j] = pltpu.async_copy(table.at[src_m.at[j]], bufs[j % NB],
                                     gsems[j % NB])
            if j >= 1:
                gh[j - 1].wait()
                sh[j - 1] = scatter_step(j - 1)
        gh[BF - 1].wait()
        sh[BF - 1] = scatter_step(BF - 1)
        for j in range(BF - NB, BF):
            sh[j].wait()
        return carry

    lax.fori_loop(0, MBLK, mblk, 0)
    plsc.subcore_barrier()

    def finish(o, sz):
        if with_deg:
            pltpu.sync_copy(dacc.at[pl.ds(pl.multiple_of(o, 8), sz)],
                            dbuf.at[pl.ds(0, sz)])

            @pl.when(c == 0)
            def _export():
                pltpu.sync_copy(dacc.at[pl.ds(pl.multiple_of(o, 8), sz)],
                                dout.at[pl.ds(pl.multiple_of(o, 8), sz)])
        else:
            pltpu.sync_copy(deg_in.at[pl.ds(pl.multiple_of(o, 8), sz)],
                            dbuf.at[pl.ds(0, sz)])
        nq = sz // CH + (1 if sz % CH else 0)
        qsizes = [min(CH, sz - q * CH) for q in range(nq)]
        hin, hout = {}, {}

        def issue_in(q):
            ro = pl.multiple_of(o + q * CH, 8)
            hin[q] = pltpu.async_copy(acc.at[pl.ds(ro, qsizes[q])],
                                      bufs[q % 2].at[pl.ds(0, qsizes[q])],
                                      gsems[q % 2])

        issue_in(0)
        for q in range(nq):
            if q + 1 < nq:
                if q >= 1:
                    hout[q - 1].wait()
                issue_in(q + 1)
            hin[q].wait()
            buf = bufs[q % 2]

            def scale(r, carry):
                dvec = dbuf[q * CH + r, :]
                inv = 1.0 / jnp.maximum(dvec, 1.0)
                for k in range(D_H // 16):
                    buf[r, pl.ds(k * 16, 16)] *= inv
                return carry

            lax.fori_loop(0, qsizes[q], scale, 0)
            obase = pl.multiple_of(c * N + o + q * CH, 8)
            hout[q] = pltpu.async_copy(buf.at[pl.ds(0, qsizes[q])],
                                       out.at[pl.ds(obase, qsizes[q])],
                                       osems[q % 2])
        for q in range(max(0, nq - 2), nq):
            hout[q].wait()

    for_stripe(finish)


def _make_sc_agg(with_deg):
    mesh = plsc.VectorSubcoreMesh(core_axis_name="c", subcore_axis_name="s")
    out_type = [jax.ShapeDtypeStruct((NC * N, D_H), jnp.float32)]
    if with_deg:
        out_type.append(jax.ShapeDtypeStruct((N, 16), jnp.float32))
    scratch = [
        pltpu.VMEM((BF, CH), jnp.int32),
        pltpu.VMEM((BF, CH), jnp.int32),
    ]
    if with_deg:
        scratch.append(pltpu.VMEM((2 * BF, CH), jnp.int32))
    scratch += [
        pltpu.VMEM((CH, D_H), jnp.float32),
        pltpu.VMEM((CH, D_H), jnp.float32),
    ]
    if not with_deg:
        scratch.append(pltpu.VMEM((CH, D_H), jnp.float32))
    if with_deg:
        scratch.append(pltpu.VMEM((CH, 16), jnp.float32))
    scratch += [
        pltpu.VMEM((S_HI, 16), jnp.float32),
        pltpu.VMEM_SHARED((N, D_H), jnp.float32),
    ]
    if with_deg:
        scratch.append(pltpu.VMEM_SHARED((N, 16), jnp.float32))
    nsem = 6 if with_deg else 8
    scratch += [pltpu.SemaphoreType.DMA] * nsem
    return pl.kernel(
        functools.partial(_sc_agg_body, with_deg),
        out_type=tuple(out_type),
        mesh=mesh,
        compiler_params=pltpu.CompilerParams(use_tc_tiling_on_sc=False),
        scratch_types=tuple(scratch),
    )


def _sage1_body(p0, p1, x, wl, wr, b, out):
    mean = p0[...] + p1[...]
    h = jnp.dot(mean, wl[...], preferred_element_type=jnp.float32)
    h += jnp.dot(x[...], wr[...], preferred_element_type=jnp.float32)
    out[...] = jnp.maximum(h + b[...], 0.0)


def _sage2_body(p0, p1, h, wl, wr, b, wdec, bdec, eps,
                mu_o, lv_o, z_o, xr_o):
    mean = p0[...] + p1[...]
    t = jnp.dot(mean, wl[...], preferred_element_type=jnp.float32)
    t += jnp.dot(h[...], wr[...], preferred_element_type=jnp.float32)
    t += b[...]
    mu = t[:, :D_Z]
    lv = t[:, D_Z:]
    z = mu + eps[...] * jnp.exp(0.5 * lv)
    mu_o[...] = mu
    lv_o[...] = lv
    z_o[...] = z
    xr_o[...] = jnp.dot(z, wdec[...], preferred_element_type=jnp.float32) + bdec[...]


def _zzt_body(zr, zc, out):
    out[...] = lax.dot_general(
        zr[...], zc[...], (((1,), (1,)), ((), ())),
        preferred_element_type=jnp.float32)


BM = 400
GB = N // BM
BA = 400
GA = N // BA


def kernel(x, edge_index, W1_l, W1_r, b1, Wmu_l, Wmu_r, bmu,
           Wlv_l, Wlv_r, blv, Wdec, bdec):
    src2d = edge_index[0].reshape(E // CH, CH)
    dst2d = edge_index[1].reshape(E // CH, CH)
    z128 = jnp.zeros((N, D_H), jnp.float32)

    agg1, deg = _make_sc_agg(True)(x, src2d, dst2d, z128)

    part_spec = pl.BlockSpec((BM, D_H), lambda i: (i, 0))
    part_spec_hi = pl.BlockSpec((BM, D_H), lambda i: (i + GB, 0))
    w_spec = pl.BlockSpec((D_H, D_H), lambda i: (0, 0))
    b_spec = pl.BlockSpec((1, D_H), lambda i: (0, 0))

    h = pl.pallas_call(
        _sage1_body,
        grid=(GB,),
        in_specs=[part_spec, part_spec_hi,
                  pl.BlockSpec((BM, D_IN), lambda i: (i, 0)),
                  w_spec, w_spec, b_spec],
        out_specs=pl.BlockSpec((BM, D_H), lambda i: (i, 0)),
        out_shape=jax.ShapeDtypeStruct((N, D_H), jnp.float32),
    )(agg1, agg1, x, W1_l, W1_r, b1.reshape(1, D_H))

    (agg2,) = _make_sc_agg(False)(h, src2d, dst2d, z128, deg)

    wl_cat = jnp.concatenate([Wmu_l, Wlv_l], axis=1)
    wr_cat = jnp.concatenate([Wmu_r, Wlv_r], axis=1)
    b_cat = jnp.concatenate([bmu, blv]).reshape(1, 2 * D_Z)
    eps = jax.random.normal(jax.random.key(42), (N, D_Z), dtype=jnp.float32)

    z_out = jax.ShapeDtypeStruct((N, D_Z), jnp.float32)
    mu, logvar, z, x_recon = pl.pallas_call(
        _sage2_body,
        grid=(GB,),
        in_specs=[part_spec, part_spec_hi,
                  pl.BlockSpec((BM, D_H), lambda i: (i, 0)),
                  w_spec, w_spec, pl.BlockSpec((1, 2 * D_Z), lambda i: (0, 0)),
                  pl.BlockSpec((D_Z, D_IN), lambda i: (0, 0)),
                  pl.BlockSpec((1, D_IN), lambda i: (0, 0)),
                  pl.BlockSpec((BM, D_Z), lambda i: (i, 0))],
        out_specs=[pl.BlockSpec((BM, D_Z), lambda i: (i, 0)),
                   pl.BlockSpec((BM, D_Z), lambda i: (i, 0)),
                   pl.BlockSpec((BM, D_Z), lambda i: (i, 0)),
                   pl.BlockSpec((BM, D_IN), lambda i: (i, 0))],
        out_shape=[z_out, z_out, z_out,
                   jax.ShapeDtypeStruct((N, D_IN), jnp.float32)],
    )(agg2, agg2, h, wl_cat, wr_cat, b_cat,
      Wdec, bdec.reshape(1, D_IN), eps)

    adj = pl.pallas_call(
        _zzt_body,
        grid=(GA,),
        in_specs=[pl.BlockSpec((BA, D_Z), lambda i: (i, 0)),
                  pl.BlockSpec((N, D_Z), lambda i: (0, 0))],
        out_specs=pl.BlockSpec((BA, N), lambda i: (i, 0)),
        out_shape=jax.ShapeDtypeStruct((N, N), jnp.float32),
    )(z, z)

    return (x_recon, adj, mu, logvar)
